# Initial kernel scaffold; baseline (speedup 1.0000x reference)
import jax
import jax.numpy as jnp
from jax.experimental import pallas as pl


def kernel(encoded, memory):
    raise NotImplementedError("write your pallas kernel here")


# speedup vs baseline: 16.6762x; 16.6762x over previous
import functools

import jax
import jax.numpy as jnp
from jax import lax
from jax.experimental import pallas as pl
from jax.experimental.pallas import tpu as pltpu
from jax.experimental.pallas import tpu_sc as plsc

_TOP_K = 5


def _normalize_body(mem_ref, out_ref):
    x = mem_ref[...]
    n = jnp.sqrt(jnp.sum(x * x, axis=1, keepdims=True))
    out_ref[...] = x / jnp.maximum(n, 1e-12)


def _normalize_mem(memory):
    m, d = memory.shape
    blk = 512
    return pl.pallas_call(
        _normalize_body,
        grid=(m // blk,),
        in_specs=[pl.BlockSpec((blk, d), lambda i: (i, 0))],
        out_specs=pl.BlockSpec((blk, d), lambda i: (i, 0)),
        out_shape=jax.ShapeDtypeStruct((m, d), jnp.float32),
    )(memory)


def _topk_body(enc_ref, memn_ref, idx_ref, *, k):
    e = enc_ref[...]
    n = jnp.sqrt(jnp.sum(e * e, axis=1, keepdims=True))
    e = e / jnp.maximum(n, 1e-12)
    scores = lax.dot_general(
        e, memn_ref[...],
        (((1,), (1,)), ((), ())),
        preferred_element_type=jnp.float32,
    )
    q, m = scores.shape
    iota = lax.broadcasted_iota(jnp.int32, (q, m), 1)
    s = scores
    cols = []
    for _ in range(k):
        mx = jnp.max(s, axis=1, keepdims=True)
        idx = jnp.min(jnp.where(s == mx, iota, jnp.int32(m)), axis=1,
                      keepdims=True)
        cols.append(idx)
        s = jnp.where(iota == idx, -jnp.inf, s)
    idx_ref[...] = jnp.concatenate(cols, axis=1)


def _topk_indices(enc2, memn, k):
    q, d = enc2.shape
    m = memn.shape[0]
    qblk = 256
    return pl.pallas_call(
        functools.partial(_topk_body, k=k),
        grid=(q // qblk,),
        in_specs=[
            pl.BlockSpec((qblk, d), lambda i: (i, 0)),
            pl.BlockSpec((m, d), lambda i: (0, 0)),
        ],
        out_specs=pl.BlockSpec((qblk, k), lambda i: (i, 0)),
        out_shape=jax.ShapeDtypeStruct((q, k), jnp.int32),
    )(enc2, memn)


def _sc_gather_mean(memory, idx_flat, enc2, k):
    q, d = enc2.shape
    info = plsc.get_sparse_core_info()
    nc, ns = info.num_cores, info.num_subcores
    nw = nc * ns
    qpw = q // nw
    cq = 8
    rows = cq * k
    nchunk = qpw // cq
    dchunks = d // 16

    mesh = plsc.VectorSubcoreMesh(core_axis_name="c", subcore_axis_name="s")

    @functools.partial(
        pl.kernel,
        mesh=mesh,
        out_type=jax.ShapeDtypeStruct((q, d), jnp.float32),
        scratch_types=[
            pltpu.VMEM((rows,), jnp.int32),
            pltpu.VMEM((rows, d), jnp.float32),
            pltpu.VMEM((cq, d), jnp.float32),
            pltpu.VMEM((cq, d), jnp.float32),
            pltpu.SemaphoreType.DMA,
        ],
    )
    def sc_kernel(mem_hbm, idx_hbm, enc_hbm, out_hbm,
                  idx_v, rows_v, enc_v, out_v, sem):
        wid = lax.axis_index("s") * nc + lax.axis_index("c")
        base_q = wid * qpw

        def chunk_body(c, carry):
            q0 = base_q + c * cq
            pltpu.sync_copy(idx_hbm.at[pl.ds(q0 * k, rows)], idx_v)
            pltpu.async_copy(mem_hbm.at[idx_v], rows_v, sem).wait()
            pltpu.sync_copy(enc_hbm.at[pl.ds(q0, cq)], enc_v)

            def d_body(dc, carry2):
                off = dc * 16
                for qq in range(cq):
                    acc = rows_v[qq * k, pl.ds(off, 16)]
                    for j in range(1, k):
                        acc = acc + rows_v[qq * k + j, pl.ds(off, 16)]
                    out_v[qq, pl.ds(off, 16)] = (
                        enc_v[qq, pl.ds(off, 16)] + acc * (1.0 / k))
                return carry2

            lax.fori_loop(0, dchunks, d_body, 0)
            pltpu.sync_copy(out_v, out_hbm.at[pl.ds(q0, cq)])
            return carry

        lax.fori_loop(0, nchunk, chunk_body, 0)

    return sc_kernel(memory, idx_flat, enc2)


def kernel(encoded, memory):
    b, l, d = encoded.shape
    enc2 = encoded.reshape(b * l, d)
    memn = _normalize_mem(memory)
    idx = _topk_indices(enc2, memn, _TOP_K)
    out = _sc_gather_mean(memory, idx.reshape(-1), enc2, _TOP_K)
    return out.reshape(b, l, d)
